# baseline (speedup 1.0000x reference)
import jax
import jax.numpy as jnp
from jax.experimental import pallas as pl
from jax.experimental.pallas import tpu as pltpu

K_CODES = 8192
D_CODE = 256

N_BLK = 1152
K_SUB = 1024
NK = K_CODES // K_SUB


def _vq_kernel(x_ref, w_ref, out_ref, mm_ref, gm_ref, ga_ref, insq_ref):
    k = pl.program_id(1)

    @pl.when(k == 0)
    def _prep():
        x = x_ref[...]
        insq_ref[...] = jnp.sum(x * x, axis=1).reshape(1, -1)

    @pl.when(k < NK)
    def _mxu():
        w_k = w_ref[...]
        mm_ref[k % 2] = jax.lax.dot_general(
            w_k, x_ref[...],
            dimension_numbers=(((1,), (1,)), ((), ())),
            preferred_element_type=jnp.float32,
        )

    @pl.when(k > 0)
    def _scan():
        j = k - 1
        mm = mm_ref[(k + 1) % 2]
        in_sqr = insq_ref[0, :]
        dist = in_sqr[None, :] - 2.0 * mm
        m_j = jnp.min(dist, axis=0)
        code_iota = jax.lax.broadcasted_iota(jnp.int32, (K_SUB, N_BLK), 0)
        a_j = jnp.min(
            jnp.where(dist == m_j[None, :], code_iota, K_CODES), axis=0)
        a_j = a_j + j * K_SUB

        @pl.when(j == 0)
        def _init():
            gm_ref[0, :] = m_j
            ga_ref[0, :] = a_j

        @pl.when(j > 0)
        def _upd():
            gm = gm_ref[0, :]
            better = m_j < gm
            ga_ref[0, :] = jnp.where(better, a_j, ga_ref[0, :])
            gm_ref[0, :] = jnp.minimum(gm, m_j)

        @pl.when(j == NK - 1)
        def _emit():
            out_ref[...] = ga_ref[...].reshape(1, 1, -1)


def kernel(z_e_x, embedding_weight):
    B, D, H, W = z_e_x.shape
    flat = jnp.transpose(z_e_x, (0, 2, 3, 1)).reshape(-1, D)
    N = flat.shape[0]
    n_tiles = N // N_BLK

    indices = pl.pallas_call(
        _vq_kernel,
        grid=(n_tiles, NK + 1),
        in_specs=[
            pl.BlockSpec((N_BLK, D), lambda n, k: (n, 0)),
            pl.BlockSpec((K_SUB, D), lambda n, k: (jnp.minimum(k, NK - 1), 0)),
        ],
        out_specs=pl.BlockSpec((1, 1, N_BLK), lambda n, k: (n, 0, 0)),
        out_shape=jax.ShapeDtypeStruct((n_tiles, 1, N_BLK), jnp.int32),
        scratch_shapes=[
            pltpu.VMEM((2, K_SUB, N_BLK), jnp.float32),
            pltpu.VMEM((1, N_BLK), jnp.float32),
            pltpu.VMEM((1, N_BLK), jnp.int32),
            pltpu.VMEM((1, N_BLK), jnp.float32),
        ],
    )(flat, embedding_weight)

    return indices.reshape(B, H, W)
